# baseline (speedup 1.0000x reference)
import jax
import jax.numpy as jnp
from jax import lax
from jax.experimental import pallas as pl
from jax.experimental.pallas import tpu as pltpu

TIME_STEPS = 64
T_BLK = 4


def _onehot_kernel(x_ref, c_ref, s_ref, out_ref, tsp_ref):
    i = pl.program_id(0)

    @pl.when(i == 0)
    def _compute_tsp():
        xv = x_ref[:][:, None]
        cv = c_ref[:][None, :]
        sv = s_ref[:][None, :]
        dist = sv * jnp.abs(xv - cv)
        tsp_ref[:] = jnp.clip(dist.astype(jnp.int32), 0, TIME_STEPS - 1)

    t_base = i * T_BLK
    tsp = tsp_ref[:]
    shape = out_ref.shape
    t_ids = lax.broadcasted_iota(jnp.int32, shape, 0) + t_base
    out_ref[:] = (t_ids == tsp[None, :, :]).astype(jnp.float32)


def kernel(x, center, scaling):
    b = x.shape[0]
    n = center.shape[0]
    grid = (TIME_STEPS // T_BLK,)
    return pl.pallas_call(
        _onehot_kernel,
        grid=grid,
        in_specs=[
            pl.BlockSpec((b,), lambda i: (0,)),
            pl.BlockSpec((n,), lambda i: (0,)),
            pl.BlockSpec((n,), lambda i: (0,)),
        ],
        out_specs=pl.BlockSpec((T_BLK, b, n), lambda i: (i, 0, 0)),
        out_shape=jax.ShapeDtypeStruct((TIME_STEPS, b, n), jnp.float32),
        scratch_shapes=[pltpu.VMEM((b, n), jnp.int32)],
    )(x, center, scaling)


# speedup vs baseline: 1.0180x; 1.0180x over previous
import jax
import jax.numpy as jnp
from jax import lax
from jax.experimental import pallas as pl
from jax.experimental.pallas import tpu as pltpu

TIME_STEPS = 64
T_BLK = 2


def _onehot_kernel(x_ref, c_ref, s_ref, out_ref, tsp_ref):
    i = pl.program_id(0)

    @pl.when(i == 0)
    def _compute_tsp():
        xv = x_ref[:][:, None]
        cv = c_ref[:][None, :]
        sv = s_ref[:][None, :]
        dist = sv * jnp.abs(xv - cv)
        tsp_ref[:] = jnp.clip(dist.astype(jnp.int32), 0, TIME_STEPS - 1)

    t_base = i * T_BLK
    tsp = tsp_ref[:]
    shape = out_ref.shape
    t_ids = lax.broadcasted_iota(jnp.int32, shape, 0) + t_base
    out_ref[:] = (t_ids == tsp[None, :, :]).astype(jnp.float32)


def kernel(x, center, scaling):
    b = x.shape[0]
    n = center.shape[0]
    grid = (TIME_STEPS // T_BLK,)
    return pl.pallas_call(
        _onehot_kernel,
        grid=grid,
        in_specs=[
            pl.BlockSpec((b,), lambda i: (0,)),
            pl.BlockSpec((n,), lambda i: (0,)),
            pl.BlockSpec((n,), lambda i: (0,)),
        ],
        out_specs=pl.BlockSpec((T_BLK, b, n), lambda i: (i, 0, 0)),
        out_shape=jax.ShapeDtypeStruct((TIME_STEPS, b, n), jnp.float32),
        scratch_shapes=[pltpu.VMEM((b, n), jnp.int32)],
    )(x, center, scaling)
